# Initial kernel scaffold; baseline (speedup 1.0000x reference)
import jax
import jax.numpy as jnp
from jax.experimental import pallas as pl


def kernel(x, Wt, bt, Wg, bg, Wr, br, Wf, bf, edge_index):
    raise NotImplementedError("write your pallas kernel here")


# speedup vs baseline: 2.9035x; 2.9035x over previous
import functools

import jax
import jax.numpy as jnp
from jax.experimental import pallas as pl
from jax.experimental.pallas import tpu as pltpu

B = 256
C = 19
T = 1024
BB = 8
R = BB * C
NCOL = 4 * C


def _shift(v, d, axis):
    if d == 0:
        return v
    n = v.shape[axis]
    dd = d % n
    if axis == 0:
        return jnp.concatenate([v[dd:, :], v[:dd, :]], axis=0)
    return jnp.concatenate([v[:, dd:], v[:, :dd]], axis=1)


def _fused_body(x_ref, wg0_ref, wg1_ref, adj_ref, wf2_ref, bg_ref, bf_ref,
                wt_ref, bt_ref, wr_ref, br_ref, out_ref):
    f32 = jnp.float32
    xb = x_ref[...]
    lane = jax.lax.broadcasted_iota(jnp.int32, (R, T), 1)

    acc0 = jnp.zeros((R, T), f32)
    acc1 = jnp.zeros((R, T), f32)
    for d in range(-4, 5):
        s = _shift(xb, d, 1)
        if d < 0:
            s = jnp.where(lane >= -d, s, 0.0)
        elif d > 0:
            s = jnp.where(lane < T - d, s, 0.0)
        acc0 = acc0 + wt_ref[0, d + 4] * s
        acc1 = acc1 + wt_ref[1, d + 4] * s
    h0 = jnp.maximum(acc0 + bt_ref[0], 0.0)
    h1 = jnp.maximum(acc1 + bt_ref[1], 0.0)

    y = (jnp.dot(h0, wg0_ref[...], preferred_element_type=f32)
         + jnp.dot(h1, wg1_ref[...], preferred_element_type=f32)
         + bg_ref[...])

    adj = adj_ref[...]
    deg = jnp.sum(adj, axis=1, keepdims=True)
    r = jax.lax.rsqrt(deg)
    ri = jax.lax.broadcasted_iota(jnp.int32, (C, C), 0)
    ci = jax.lax.broadcasted_iota(jnp.int32, (C, C), 1)
    dmat = jnp.where(ri == ci, jnp.broadcast_to(r, (C, C)), 0.0)
    a = jnp.dot(dmat, jnp.dot(adj, dmat, preferred_element_type=f32),
                preferred_element_type=f32)
    arows = jnp.concatenate([a] * BB, axis=0)
    afull = jnp.concatenate([arows] * BB, axis=1)
    rri = jax.lax.broadcasted_iota(jnp.int32, (R, R), 0) // C
    cci = jax.lax.broadcasted_iota(jnp.int32, (R, R), 1) // C
    akron = jnp.where(rri == cci, afull, 0.0)
    xs = jnp.maximum(jnp.dot(akron, y, preferred_element_type=f32), 0.0)

    cpos = jax.lax.broadcasted_iota(jnp.int32, (R, T), 0) % C
    racc = jnp.zeros((R, T), f32)
    for dc in (-1, 0, 1):
        s = _shift(xs, dc, 0)
        if dc < 0:
            s = jnp.where(cpos >= -dc, s, 0.0)
        elif dc > 0:
            s = jnp.where(cpos < C - dc, s, 0.0)
        for dt in (-1, 0, 1):
            s2 = _shift(s, dt, 1)
            if dt < 0:
                s2 = jnp.where(lane >= -dt, s2, 0.0)
            elif dt > 0:
                s2 = jnp.where(lane < T - dt, s2, 0.0)
            racc = racc + wr_ref[dc + 1, dt + 1] * s2
    xres = jnp.maximum(xs + racc + br_ref[0], 0.0)

    p = jnp.dot(xres, wf2_ref[...], preferred_element_type=f32)
    rp = jax.lax.broadcasted_iota(jnp.int32, (R, NCOL), 0) % C
    cp = jax.lax.broadcasted_iota(jnp.int32, (R, NCOL), 1) // 4
    dsel = jnp.where(rp == cp, p, 0.0)
    s4r = jax.lax.broadcasted_iota(jnp.int32, (NCOL, 4), 0) % 4
    s4c = jax.lax.broadcasted_iota(jnp.int32, (NCOL, 4), 1)
    sel4 = jnp.where(s4r == s4c, 1.0, 0.0).astype(f32)
    q = jnp.dot(dsel, sel4, preferred_element_type=f32)
    gr = jax.lax.broadcasted_iota(jnp.int32, (BB, R), 0)
    gc = jax.lax.broadcasted_iota(jnp.int32, (BB, R), 1) // C
    gsum = jnp.where(gr == gc, 1.0, 0.0).astype(f32)
    out_ref[...] = jnp.dot(gsum, q, preferred_element_type=f32) + bf_ref[...]


@functools.partial(jax.jit, static_argnames=())
def _fused(xr, wg0, wg1, adj, wf2, bg2, bf2, wt2, bt, wr2, br):
    grid = (B // BB,)
    return pl.pallas_call(
        _fused_body,
        grid=grid,
        in_specs=[
            pl.BlockSpec((R, T), lambda i: (i, 0)),
            pl.BlockSpec((T, T), lambda i: (0, 0)),
            pl.BlockSpec((T, T), lambda i: (0, 0)),
            pl.BlockSpec((C, C), lambda i: (0, 0)),
            pl.BlockSpec((T, NCOL), lambda i: (0, 0)),
            pl.BlockSpec((1, T), lambda i: (0, 0)),
            pl.BlockSpec((1, 4), lambda i: (0, 0)),
            pl.BlockSpec(memory_space=pltpu.SMEM),
            pl.BlockSpec(memory_space=pltpu.SMEM),
            pl.BlockSpec(memory_space=pltpu.SMEM),
            pl.BlockSpec(memory_space=pltpu.SMEM),
        ],
        out_specs=pl.BlockSpec((BB, 4), lambda i: (i, 0)),
        out_shape=jax.ShapeDtypeStruct((B, 4), jnp.float32),
        compiler_params=pltpu.CompilerParams(
            dimension_semantics=("arbitrary",),
        ),
    )(xr, wg0, wg1, adj, wf2, bg2, bf2, wt2, bt, wr2, br)


def kernel(x, Wt, bt, Wg, bg, Wr, br, Wf, bf, edge_index):
    xr = x.reshape(B * C, T)
    wt2 = Wt.reshape(2, 9)
    wg0 = Wg[:T, :]
    wg1 = Wg[T:, :]
    wr2 = Wr.reshape(3, 3)
    wf2 = jnp.transpose(Wf.reshape(C, T, 4), (1, 0, 2)).reshape(T, NCOL)
    bg2 = bg.reshape(1, T)
    bf2 = bf.reshape(1, 4)
    self_loop = jnp.arange(C, dtype=edge_index.dtype)
    src = jnp.concatenate([edge_index[0], self_loop])
    dst = jnp.concatenate([edge_index[1], self_loop])
    adj = jnp.zeros((C, C), jnp.float32).at[dst, src].add(1.0)
    return _fused(xr, wg0, wg1, adj, wf2, bg2, bf2, wt2, bt, wr2, br)


# speedup vs baseline: 3.1130x; 1.0721x over previous
import functools

import jax
import jax.numpy as jnp
from jax.experimental import pallas as pl
from jax.experimental.pallas import tpu as pltpu

B = 256
C = 19
T = 1024
BB = 16
R = BB * C
NCOL = 4 * C


def _shift(v, d, axis):
    if d == 0:
        return v
    n = v.shape[axis]
    dd = d % n
    if axis == 0:
        return jnp.concatenate([v[dd:, :], v[:dd, :]], axis=0)
    return jnp.concatenate([v[:, dd:], v[:, :dd]], axis=1)


def _fused_body(x_ref, wg_ref, adj_ref, wf2_ref, bg_ref, bf_ref,
                wt_ref, bt_ref, wr_ref, br_ref, out_ref):
    f32 = jnp.float32
    xb = x_ref[...]
    lane = jax.lax.broadcasted_iota(jnp.int32, (R, T), 1)

    acc0 = jnp.zeros((R, T), f32)
    acc1 = jnp.zeros((R, T), f32)
    for d in range(-4, 5):
        s = _shift(xb, d, 1)
        if d < 0:
            s = jnp.where(lane >= -d, s, 0.0)
        elif d > 0:
            s = jnp.where(lane < T - d, s, 0.0)
        acc0 = acc0 + wt_ref[0, d + 4] * s
        acc1 = acc1 + wt_ref[1, d + 4] * s
    h0 = jnp.maximum(acc0 + bt_ref[0], 0.0)
    h1 = jnp.maximum(acc1 + bt_ref[1], 0.0)

    y = (jnp.dot(h0, wg_ref[:T, :], preferred_element_type=f32)
         + jnp.dot(h1, wg_ref[T:, :], preferred_element_type=f32)
         + bg_ref[...])

    adj = adj_ref[...]
    deg = jnp.sum(adj, axis=1, keepdims=True)
    r = jax.lax.rsqrt(deg)
    ri = jax.lax.broadcasted_iota(jnp.int32, (C, C), 0)
    ci = jax.lax.broadcasted_iota(jnp.int32, (C, C), 1)
    dmat = jnp.where(ri == ci, jnp.broadcast_to(r, (C, C)), 0.0)
    a = jnp.dot(dmat, jnp.dot(adj, dmat, preferred_element_type=f32),
                preferred_element_type=f32)
    arows = jnp.concatenate([a] * BB, axis=0)
    afull = jnp.concatenate([arows] * BB, axis=1)
    rri = jax.lax.broadcasted_iota(jnp.int32, (R, R), 0) // C
    cci = jax.lax.broadcasted_iota(jnp.int32, (R, R), 1) // C
    akron = jnp.where(rri == cci, afull, 0.0)
    xs = jnp.maximum(jnp.dot(akron, y, preferred_element_type=f32), 0.0)

    cpos = jax.lax.broadcasted_iota(jnp.int32, (R, T), 0) % C
    racc = jnp.zeros((R, T), f32)
    for dc in (-1, 0, 1):
        s = _shift(xs, dc, 0)
        if dc < 0:
            s = jnp.where(cpos >= -dc, s, 0.0)
        elif dc > 0:
            s = jnp.where(cpos < C - dc, s, 0.0)
        for dt in (-1, 0, 1):
            s2 = _shift(s, dt, 1)
            if dt < 0:
                s2 = jnp.where(lane >= -dt, s2, 0.0)
            elif dt > 0:
                s2 = jnp.where(lane < T - dt, s2, 0.0)
            racc = racc + wr_ref[dc + 1, dt + 1] * s2
    xres = jnp.maximum(xs + racc + br_ref[0], 0.0)

    p = jnp.dot(xres, wf2_ref[...], preferred_element_type=f32)
    rp = jax.lax.broadcasted_iota(jnp.int32, (R, NCOL), 0) % C
    cp = jax.lax.broadcasted_iota(jnp.int32, (R, NCOL), 1) // 4
    dsel = jnp.where(rp == cp, p, 0.0)
    s4r = jax.lax.broadcasted_iota(jnp.int32, (NCOL, 4), 0) % 4
    s4c = jax.lax.broadcasted_iota(jnp.int32, (NCOL, 4), 1)
    sel4 = jnp.where(s4r == s4c, 1.0, 0.0).astype(f32)
    q = jnp.dot(dsel, sel4, preferred_element_type=f32)
    gr = jax.lax.broadcasted_iota(jnp.int32, (BB, R), 0)
    gc = jax.lax.broadcasted_iota(jnp.int32, (BB, R), 1) // C
    gsum = jnp.where(gr == gc, 1.0, 0.0).astype(f32)
    out_ref[...] = jnp.dot(gsum, q, preferred_element_type=f32) + bf_ref[...]


@functools.partial(jax.jit, static_argnames=())
def _fused(xr, wg, adj, wf2, bg2, bf2, wt2, bt, wr2, br):
    grid = (B // BB,)
    return pl.pallas_call(
        _fused_body,
        grid=grid,
        in_specs=[
            pl.BlockSpec((R, T), lambda i: (i, 0)),
            pl.BlockSpec((2 * T, T), lambda i: (0, 0)),
            pl.BlockSpec((C, C), lambda i: (0, 0)),
            pl.BlockSpec((T, NCOL), lambda i: (0, 0)),
            pl.BlockSpec((1, T), lambda i: (0, 0)),
            pl.BlockSpec((1, 4), lambda i: (0, 0)),
            pl.BlockSpec(memory_space=pltpu.SMEM),
            pl.BlockSpec(memory_space=pltpu.SMEM),
            pl.BlockSpec(memory_space=pltpu.SMEM),
            pl.BlockSpec(memory_space=pltpu.SMEM),
        ],
        out_specs=pl.BlockSpec((BB, 4), lambda i: (i, 0)),
        out_shape=jax.ShapeDtypeStruct((B, 4), jnp.float32),
        compiler_params=pltpu.CompilerParams(
            dimension_semantics=("arbitrary",),
        ),
    )(xr, wg, adj, wf2, bg2, bf2, wt2, bt, wr2, br)


def kernel(x, Wt, bt, Wg, bg, Wr, br, Wf, bf, edge_index):
    xr = x.reshape(B * C, T)
    wt2 = Wt.reshape(2, 9)
    wr2 = Wr.reshape(3, 3)
    wf2 = jnp.transpose(Wf.reshape(C, T, 4), (1, 0, 2)).reshape(T, NCOL)
    bg2 = bg.reshape(1, T)
    bf2 = bf.reshape(1, 4)
    self_loop = jnp.arange(C, dtype=edge_index.dtype)
    src = jnp.concatenate([edge_index[0], self_loop])
    dst = jnp.concatenate([edge_index[1], self_loop])
    adj = jnp.zeros((C, C), jnp.float32).at[dst, src].add(1.0)
    return _fused(xr, Wg, adj, wf2, bg2, bf2, wt2, bt, wr2, br)


# speedup vs baseline: 3.7409x; 1.2017x over previous
import functools

import jax
import jax.numpy as jnp
from jax.experimental import pallas as pl
from jax.experimental.pallas import tpu as pltpu

B = 256
C = 19
T = 1024
BB = 16
R = BB * C
NCOL = 4 * C


def _shift(v, d, axis):
    if d == 0:
        return v
    n = v.shape[axis]
    dd = d % n
    if axis == 0:
        return jnp.concatenate([v[dd:, :], v[:dd, :]], axis=0)
    return jnp.concatenate([v[:, dd:], v[:, :dd]], axis=1)


def _fused_body(x_ref, wg_ref, adj_ref, wf2_ref, bg_ref, bf_ref,
                wt_ref, bt_ref, wr_ref, br_ref, out_ref, k_ref, ak_ref):
    f32 = jnp.float32

    @pl.when(pl.program_id(0) == 0)
    def _build_constants():
        kri = jax.lax.broadcasted_iota(jnp.int32, (T, 2 * T), 0)
        kci = jax.lax.broadcasted_iota(jnp.int32, (T, 2 * T), 1)
        kd = kri - (kci % T)
        kacc = jnp.zeros((T, 2 * T), f32)
        for d in range(-4, 5):
            w = jnp.where(kci < T, wt_ref[0, d + 4], wt_ref[1, d + 4])
            kacc = kacc + jnp.where(kd == d, w, 0.0)
        k_ref[...] = kacc
        adj = adj_ref[...]
        deg = jnp.sum(adj, axis=1, keepdims=True)
        r = jax.lax.rsqrt(deg)
        ri = jax.lax.broadcasted_iota(jnp.int32, (C, C), 0)
        ci = jax.lax.broadcasted_iota(jnp.int32, (C, C), 1)
        dmat = jnp.where(ri == ci, jnp.broadcast_to(r, (C, C)), 0.0)
        a = jnp.dot(dmat, jnp.dot(adj, dmat, preferred_element_type=f32),
                    preferred_element_type=f32)
        arows = jnp.concatenate([a] * BB, axis=0)
        afull = jnp.concatenate([arows] * BB, axis=1)
        rri = jax.lax.broadcasted_iota(jnp.int32, (R, R), 0) // C
        cci = jax.lax.broadcasted_iota(jnp.int32, (R, R), 1) // C
        ak_ref[...] = jnp.where(rri == cci, afull, 0.0)

    xb = x_ref[...]
    lane = jax.lax.broadcasted_iota(jnp.int32, (R, T), 1)

    lane2 = jax.lax.broadcasted_iota(jnp.int32, (1, 2 * T), 1)
    btsel = jnp.where(lane2 < T, bt_ref[0], bt_ref[1])
    h = jnp.maximum(
        jnp.dot(xb, k_ref[...], preferred_element_type=f32) + btsel, 0.0)

    y = jnp.dot(h, wg_ref[...], preferred_element_type=f32) + bg_ref[...]

    xs = jnp.maximum(jnp.dot(ak_ref[...], y, preferred_element_type=f32), 0.0)

    cpos = jax.lax.broadcasted_iota(jnp.int32, (R, T), 0) % C
    racc = jnp.zeros((R, T), f32)
    for dc in (-1, 0, 1):
        s = _shift(xs, dc, 0)
        if dc < 0:
            s = jnp.where(cpos >= -dc, s, 0.0)
        elif dc > 0:
            s = jnp.where(cpos < C - dc, s, 0.0)
        for dt in (-1, 0, 1):
            s2 = _shift(s, dt, 1)
            if dt < 0:
                s2 = jnp.where(lane >= -dt, s2, 0.0)
            elif dt > 0:
                s2 = jnp.where(lane < T - dt, s2, 0.0)
            racc = racc + wr_ref[dc + 1, dt + 1] * s2
    xres = jnp.maximum(xs + racc + br_ref[0], 0.0)

    p = jnp.dot(xres, wf2_ref[...], preferred_element_type=f32)
    rp = jax.lax.broadcasted_iota(jnp.int32, (R, NCOL), 0) % C
    cp = jax.lax.broadcasted_iota(jnp.int32, (R, NCOL), 1) // 4
    dsel = jnp.where(rp == cp, p, 0.0)
    s4r = jax.lax.broadcasted_iota(jnp.int32, (NCOL, 4), 0) % 4
    s4c = jax.lax.broadcasted_iota(jnp.int32, (NCOL, 4), 1)
    sel4 = jnp.where(s4r == s4c, 1.0, 0.0).astype(f32)
    q = jnp.dot(dsel, sel4, preferred_element_type=f32)
    gr = jax.lax.broadcasted_iota(jnp.int32, (BB, R), 0)
    gc = jax.lax.broadcasted_iota(jnp.int32, (BB, R), 1) // C
    gsum = jnp.where(gr == gc, 1.0, 0.0).astype(f32)
    out_ref[...] = jnp.dot(gsum, q, preferred_element_type=f32) + bf_ref[...]


@functools.partial(jax.jit, static_argnames=())
def _fused(xr, wg, adj, wf2, bg2, bf2, wt2, bt, wr2, br):
    grid = (B // BB,)
    return pl.pallas_call(
        _fused_body,
        grid=grid,
        in_specs=[
            pl.BlockSpec((R, T), lambda i: (i, 0)),
            pl.BlockSpec((2 * T, T), lambda i: (0, 0)),
            pl.BlockSpec((C, C), lambda i: (0, 0)),
            pl.BlockSpec((T, NCOL), lambda i: (0, 0)),
            pl.BlockSpec((1, T), lambda i: (0, 0)),
            pl.BlockSpec((1, 4), lambda i: (0, 0)),
            pl.BlockSpec(memory_space=pltpu.SMEM),
            pl.BlockSpec(memory_space=pltpu.SMEM),
            pl.BlockSpec(memory_space=pltpu.SMEM),
            pl.BlockSpec(memory_space=pltpu.SMEM),
        ],
        out_specs=pl.BlockSpec((BB, 4), lambda i: (i, 0)),
        out_shape=jax.ShapeDtypeStruct((B, 4), jnp.float32),
        scratch_shapes=[
            pltpu.VMEM((T, 2 * T), jnp.float32),
            pltpu.VMEM((R, R), jnp.float32),
        ],
        compiler_params=pltpu.CompilerParams(
            dimension_semantics=("arbitrary",),
        ),
    )(xr, wg, adj, wf2, bg2, bf2, wt2, bt, wr2, br)


def kernel(x, Wt, bt, Wg, bg, Wr, br, Wf, bf, edge_index):
    xr = x.reshape(B * C, T)
    wt2 = Wt.reshape(2, 9)
    wr2 = Wr.reshape(3, 3)
    wf2 = jnp.transpose(Wf.reshape(C, T, 4), (1, 0, 2)).reshape(T, NCOL)
    bg2 = bg.reshape(1, T)
    bf2 = bf.reshape(1, 4)
    self_loop = jnp.arange(C, dtype=edge_index.dtype)
    src = jnp.concatenate([edge_index[0], self_loop])
    dst = jnp.concatenate([edge_index[1], self_loop])
    adj = jnp.zeros((C, C), jnp.float32).at[dst, src].add(1.0)
    return _fused(xr, Wg, adj, wf2, bg2, bf2, wt2, bt, wr2, br)
